# baseline (speedup 1.0000x reference)
import functools

import jax
import jax.numpy as jnp
from jax import lax
from jax.experimental import pallas as pl
from jax.experimental.pallas import tpu as pltpu
from jax.experimental.pallas import tpu_sc as plsc

B, L, H = 4096, 200, 128
NA, ND, NSTEP = 7, 41, 15
V = 4312
EPS = 1e-12
BL = B * L

def _table_body(a_ref, d_ref, s_ref, g_ref, bt_ref, out_ref):
    c = lax.broadcasted_iota(jnp.int32, (V, 1), 0)
    aid = c // (ND * NSTEP)
    r = c % (ND * NSTEP)
    did = r // NSTEP
    sid = r % NSTEP

    def oh(ids, n):
        return (ids == lax.broadcasted_iota(jnp.int32, (1, n), 1)).astype(jnp.float32)

    x = (jnp.dot(oh(aid, NA), a_ref[:], preferred_element_type=jnp.float32)
         + jnp.dot(oh(did, ND), d_ref[:], preferred_element_type=jnp.float32)
         + jnp.dot(oh(sid, NSTEP), s_ref[:], preferred_element_type=jnp.float32))
    mean = jnp.mean(x, axis=1, keepdims=True)
    xc = x - mean
    var = jnp.mean(xc * xc, axis=1, keepdims=True)
    out_ref[:] = xc * lax.rsqrt(var + EPS) * g_ref[:] + bt_ref[:]


def _build_table(at, dt, st, gamma, beta):
    return pl.pallas_call(
        _table_body,
        out_shape=jax.ShapeDtypeStruct((V, H), jnp.float32),
    )(at, dt, st, gamma.reshape(1, H), beta.reshape(1, H))


NCORES, NSUB = 2, 16
NW = NCORES * NSUB
CH = 128
ROWS_PER_W = BL // NW
CHUNKS = ROWS_PER_W // CH
PAIRS = CHUNKS // 2
BATCH_PER_W = B // NW
NQ = 4
BH = BATCH_PER_W // NQ
QUARTER = BH * L
QPAIRS = PAIRS // NQ


@functools.cache
def _make_gather():
    mesh = plsc.VectorSubcoreMesh(core_axis_name="c", subcore_axis_name="s",
                                  num_cores=NCORES)
    return functools.partial(
        pl.kernel,
        mesh=mesh,
        out_type=jax.ShapeDtypeStruct((BL, H), jnp.float32),
        scratch_types=[
            pltpu.VMEM_SHARED((V, H), jnp.float32),
            pltpu.VMEM((BH, L), jnp.int32),
            pltpu.VMEM((BH, L), jnp.int32),
            pltpu.VMEM((BH, L), jnp.int32),
            pltpu.VMEM((2 * QUARTER,), jnp.int32),
            pltpu.VMEM((CH, H), jnp.float32),
            pltpu.VMEM((CH, H), jnp.float32),
            pltpu.VMEM((CH, H), jnp.float32),
            pltpu.SemaphoreType.DMA,
            pltpu.SemaphoreType.DMA,
            pltpu.SemaphoreType.DMA,
            pltpu.SemaphoreType.DMA,
            pltpu.SemaphoreType.DMA,
            pltpu.SemaphoreType.DMA,
            pltpu.SemaphoreType.DMA,
        ],
    )(_gather_body)


def _gather_body(table_hbm, a_hbm, d_hbm, s_hbm, out_hbm,
                 table_sp, sa, sd, ss, cidx, buf0, buf1, buf2,
                 gs0, gs1, gs2, ws0, ws1, ws2, ssem):
    sid = lax.axis_index("s")
    wid = sid * NCORES + lax.axis_index("c")
    row0 = wid * ROWS_PER_W
    bufs = (buf0, buf1, buf2)
    gsem = (gs0, gs1, gs2)
    wsem = (ws0, ws1, ws2)

    @pl.when(sid == 0)
    def _():
        pltpu.sync_copy(table_hbm, table_sp)

    iota16 = jnp.arange(16, dtype=jnp.int32)

    def stage_quarter(q, sync=False):
        r0 = wid * BATCH_PER_W + q * BH
        if sync:
            pltpu.sync_copy(a_hbm.at[pl.ds(r0, BH)], sa)
            pltpu.sync_copy(d_hbm.at[pl.ds(r0, BH)], sd)
            pltpu.sync_copy(s_hbm.at[pl.ds(r0, BH)], ss)
        else:
            pltpu.async_copy(a_hbm.at[pl.ds(r0, BH)], sa, ssem)
            pltpu.async_copy(d_hbm.at[pl.ds(r0, BH)], sd, ssem)
            pltpu.async_copy(s_hbm.at[pl.ds(r0, BH)], ss, ssem)

    def drain_stage():
        for buf in (sa, sd, ss):
            pltpu.make_async_copy(a_hbm.at[pl.ds(0, BH)], buf, ssem).wait()

    _COLS = tuple(range(0, L - 16, 16)) + (L - 16,)

    def compute_quarter(q):
        base = (q % 2) * QUARTER

        def crow(r, carry):
            for c in _COLS:
                av = sa[r, pl.ds(c, 16)]
                dv = sd[r, pl.ds(c, 16)]
                sv = ss[r, pl.ds(c, 16)]
                cc = av * (ND * NSTEP) + dv * NSTEP + sv
                cidx[pl.ds(base + r * L + c, 16)] = cc
            return carry

        lax.fori_loop(0, BH, crow, 0)

    def start_g(g, b):
        off = lax.rem(g * CH, 2 * QUARTER)
        pltpu.async_copy(table_sp.at[cidx.at[pl.ds(off, CH)]],
                         bufs[b], gsem[b])

    def drain_g(b):
        pltpu.make_async_copy(out_hbm.at[pl.ds(0, CH)], bufs[b], gsem[b]).wait()

    def start_w(g, b):
        pltpu.async_copy(bufs[b], out_hbm.at[pl.ds(row0 + g * CH, CH)], wsem[b])

    def drain_w(b):
        pltpu.make_async_copy(bufs[b], out_hbm.at[pl.ds(0, CH)], wsem[b]).wait()

    stage_quarter(0, sync=True)
    compute_quarter(0)
    plsc.subcore_barrier()

    start_g(0, 0)
    start_g(1, 1)
    stage_quarter(1)
    drain_stage()
    compute_quarter(1)
    stage_quarter(2)
    start_g(2, 2)
    drain_g(0)
    start_w(0, 0)
    drain_g(1)
    start_w(1, 1)

    def body(i, carry):
        for j in range(3):
            g = 3 * i + j
            drain_w(j)
            start_g(g, j)
            drain_g((j + 2) % 3)
            start_w(g - 1, (j + 2) % 3)
        return carry

    lax.fori_loop(1, 33, body, 0)
    drain_stage()
    compute_quarter(2)
    stage_quarter(3)
    lax.fori_loop(33, 50, body, 0)
    drain_stage()
    compute_quarter(3)
    lax.fori_loop(50, 66, body, 0)
    drain_w(0)
    start_g(198, 0)
    drain_g(2)
    start_w(197, 2)
    drain_w(1)
    start_g(199, 1)
    drain_g(0)
    start_w(198, 0)
    drain_g(1)
    start_w(199, 1)
    drain_w(2)
    drain_w(0)
    drain_w(1)


def kernel(act_seq, act_dist, act_step, action_table, distance_table,
           step_table, gamma, beta):
    table = _build_table(action_table, distance_table, step_table, gamma, beta)
    out = _make_gather()(table, act_seq, act_dist, act_step)
    return out.reshape(B, L, H)


# speedup vs baseline: 1.0431x; 1.0431x over previous
import functools

import jax
import jax.numpy as jnp
from jax import lax
from jax.experimental import pallas as pl
from jax.experimental.pallas import tpu as pltpu
from jax.experimental.pallas import tpu_sc as plsc

B, L, H = 4096, 200, 128
NA, ND, NSTEP = 7, 41, 15
V = 4312
EPS = 1e-12
BL = B * L

def _table_body(a_ref, d_ref, s_ref, g_ref, bt_ref, out_ref):
    c = lax.broadcasted_iota(jnp.int32, (V, 1), 0)
    aid = c // (ND * NSTEP)
    r = c % (ND * NSTEP)
    did = r // NSTEP
    sid = r % NSTEP

    def oh(ids, n):
        return (ids == lax.broadcasted_iota(jnp.int32, (1, n), 1)).astype(jnp.float32)

    x = (jnp.dot(oh(aid, NA), a_ref[:], preferred_element_type=jnp.float32)
         + jnp.dot(oh(did, ND), d_ref[:], preferred_element_type=jnp.float32)
         + jnp.dot(oh(sid, NSTEP), s_ref[:], preferred_element_type=jnp.float32))
    mean = jnp.mean(x, axis=1, keepdims=True)
    xc = x - mean
    var = jnp.mean(xc * xc, axis=1, keepdims=True)
    out_ref[:] = xc * lax.rsqrt(var + EPS) * g_ref[:] + bt_ref[:]


def _build_table(at, dt, st, gamma, beta):
    return pl.pallas_call(
        _table_body,
        out_shape=jax.ShapeDtypeStruct((V, H), jnp.float32),
    )(at, dt, st, gamma.reshape(1, H), beta.reshape(1, H))


NCORES, NSUB = 2, 16
NW = NCORES * NSUB
CH = 256
ROWS_PER_W = BL // NW
CHUNKS = ROWS_PER_W // CH
PAIRS = CHUNKS // 2
BATCH_PER_W = B // NW
NE = 8
BH = BATCH_PER_W // NE
EIGHTH = BH * L
SLOT = 2 * EIGHTH


@functools.cache
def _make_gather():
    mesh = plsc.VectorSubcoreMesh(core_axis_name="c", subcore_axis_name="s",
                                  num_cores=NCORES)
    return functools.partial(
        pl.kernel,
        mesh=mesh,
        out_type=jax.ShapeDtypeStruct((BL, H), jnp.float32),
        scratch_types=[
            pltpu.VMEM_SHARED((V, H), jnp.float32),
            pltpu.VMEM((BH, L), jnp.int32),
            pltpu.VMEM((BH, L), jnp.int32),
            pltpu.VMEM((BH, L), jnp.int32),
            pltpu.VMEM((2 * SLOT,), jnp.int32),
            pltpu.VMEM((CH, H), jnp.float32),
            pltpu.VMEM((CH, H), jnp.float32),
            pltpu.SemaphoreType.DMA,
            pltpu.SemaphoreType.DMA,
            pltpu.SemaphoreType.DMA,
            pltpu.SemaphoreType.DMA,
            pltpu.SemaphoreType.DMA,
        ],
    )(_gather_body)


def _gather_body(table_hbm, a_hbm, d_hbm, s_hbm, out_hbm,
                 table_sp, sa, sd, ss, cidx, buf0, buf1,
                 gs0, gs1, ws0, ws1, ssem):
    sid = lax.axis_index("s")
    wid = sid * NCORES + lax.axis_index("c")
    row0 = wid * ROWS_PER_W
    bufs = (buf0, buf1)
    gsem = (gs0, gs1)
    wsem = (ws0, ws1)

    @pl.when(sid == 0)
    def _():
        pltpu.sync_copy(table_hbm, table_sp)

    iota16 = jnp.arange(16, dtype=jnp.int32)

    def stage_eighth(e, sync=False):
        r0 = wid * BATCH_PER_W + e * BH
        if sync:
            pltpu.sync_copy(a_hbm.at[pl.ds(r0, BH)], sa)
            pltpu.sync_copy(d_hbm.at[pl.ds(r0, BH)], sd)
            pltpu.sync_copy(s_hbm.at[pl.ds(r0, BH)], ss)
        else:
            pltpu.async_copy(a_hbm.at[pl.ds(r0, BH)], sa, ssem)
            pltpu.async_copy(d_hbm.at[pl.ds(r0, BH)], sd, ssem)
            pltpu.async_copy(s_hbm.at[pl.ds(r0, BH)], ss, ssem)

    def drain_stage():
        for buf in (sa, sd, ss):
            pltpu.make_async_copy(a_hbm.at[pl.ds(0, BH)], buf, ssem).wait()

    _COLS = tuple(range(0, L - 16, 16)) + (L - 16,)

    def compute_eighth(e):
        base = ((e // 2) % 2) * SLOT + (e % 2) * EIGHTH

        def crow(r, carry):
            for c in _COLS:
                av = sa[r, pl.ds(c, 16)]
                dv = sd[r, pl.ds(c, 16)]
                sv = ss[r, pl.ds(c, 16)]
                cc = av * (ND * NSTEP) + dv * NSTEP + sv
                cidx[pl.ds(base + r * L + c, 16)] = cc
            return carry

        lax.fori_loop(0, BH, crow, 0)

    def step_eighth(e):
        drain_stage()
        compute_eighth(e)
        if e + 1 < NE:
            stage_eighth(e + 1)

    def start_g(g, b):
        off = lax.rem(g * CH, 2 * SLOT)
        pltpu.async_copy(table_sp.at[cidx.at[pl.ds(off, CH)]],
                         bufs[b], gsem[b])

    def drain_g(b):
        pltpu.make_async_copy(out_hbm.at[pl.ds(0, CH)], bufs[b], gsem[b]).wait()

    def start_w(g, b):
        pltpu.async_copy(bufs[b], out_hbm.at[pl.ds(row0 + g * CH, CH)], wsem[b])

    def drain_w(b):
        pltpu.make_async_copy(bufs[b], out_hbm.at[pl.ds(0, CH)], wsem[b]).wait()

    stage_eighth(0, sync=True)
    compute_eighth(0)
    stage_eighth(1)
    plsc.subcore_barrier()

    start_g(0, 0)
    start_g(1, 1)
    step_eighth(1)
    step_eighth(2)
    step_eighth(3)
    drain_g(0)
    start_w(0, 0)

    def body(i, carry):
        drain_w(0)
        start_g(2 * i, 0)
        drain_g(1)
        start_w(2 * i - 1, 1)
        drain_w(1)
        start_g(2 * i + 1, 1)
        drain_g(0)
        start_w(2 * i, 0)
        return carry

    lax.fori_loop(1, 25, body, 0)
    step_eighth(4)
    step_eighth(5)
    lax.fori_loop(25, 37, body, 0)
    step_eighth(6)
    step_eighth(7)
    lax.fori_loop(37, PAIRS, body, 0)
    drain_g(1)
    start_w(2 * PAIRS - 1, 1)
    drain_w(0)
    drain_w(1)


def kernel(act_seq, act_dist, act_step, action_table, distance_table,
           step_table, gamma, beta):
    table = _build_table(action_table, distance_table, step_table, gamma, beta)
    out = _make_gather()(table, act_seq, act_dist, act_step)
    return out.reshape(B, L, H)


# speedup vs baseline: 1.0731x; 1.0288x over previous
import functools

import jax
import jax.numpy as jnp
from jax import lax
from jax.experimental import pallas as pl
from jax.experimental.pallas import tpu as pltpu
from jax.experimental.pallas import tpu_sc as plsc

B, L, H = 4096, 200, 128
NA, ND, NSTEP = 7, 41, 15
V = 4312
EPS = 1e-12
BL = B * L

def _table_body(a_ref, d_ref, s_ref, g_ref, bt_ref, out_ref):
    c = lax.broadcasted_iota(jnp.int32, (V, 1), 0)
    aid = c // (ND * NSTEP)
    r = c % (ND * NSTEP)
    did = r // NSTEP
    sid = r % NSTEP

    def oh(ids, n):
        return (ids == lax.broadcasted_iota(jnp.int32, (1, n), 1)).astype(jnp.float32)

    x = (jnp.dot(oh(aid, NA), a_ref[:], preferred_element_type=jnp.float32)
         + jnp.dot(oh(did, ND), d_ref[:], preferred_element_type=jnp.float32)
         + jnp.dot(oh(sid, NSTEP), s_ref[:], preferred_element_type=jnp.float32))
    mean = jnp.mean(x, axis=1, keepdims=True)
    xc = x - mean
    var = jnp.mean(xc * xc, axis=1, keepdims=True)
    out_ref[:] = xc * lax.rsqrt(var + EPS) * g_ref[:] + bt_ref[:]


def _build_table(at, dt, st, gamma, beta):
    return pl.pallas_call(
        _table_body,
        out_shape=jax.ShapeDtypeStruct((V, H), jnp.float32),
    )(at, dt, st, gamma.reshape(1, H), beta.reshape(1, H))


NCORES, NSUB = 2, 16
NW = NCORES * NSUB
CH = 256
ROWS_PER_W = BL // NW
CHUNKS = ROWS_PER_W // CH
PAIRS = CHUNKS // 2
BATCH_PER_W = B // NW
NE = 8
BH = BATCH_PER_W // NE
EIGHTH = BH * L
SLOT = 2 * EIGHTH


@functools.cache
def _make_gather():
    mesh = plsc.VectorSubcoreMesh(core_axis_name="c", subcore_axis_name="s",
                                  num_cores=NCORES)
    return functools.partial(
        pl.kernel,
        mesh=mesh,
        out_type=jax.ShapeDtypeStruct((BL, H), jnp.float32),
        scratch_types=[
            pltpu.VMEM_SHARED((V, H), jnp.float32),
            pltpu.VMEM((BH, L), jnp.int32),
            pltpu.VMEM((BH, L), jnp.int32),
            pltpu.VMEM((BH, L), jnp.int32),
            pltpu.VMEM((2 * SLOT,), jnp.int32),
            pltpu.VMEM((CH, H), jnp.float32),
            pltpu.VMEM((CH, H), jnp.float32),
            pltpu.SemaphoreType.DMA,
            pltpu.SemaphoreType.DMA,
            pltpu.SemaphoreType.DMA,
            pltpu.SemaphoreType.DMA,
            pltpu.SemaphoreType.DMA,
        ],
    )(_gather_body)


def _gather_body(table_hbm, a_hbm, d_hbm, s_hbm, out_hbm,
                 table_sp, sa, sd, ss, cidx, buf0, buf1,
                 gs0, gs1, ws0, ws1, ssem):
    sid = lax.axis_index("s")
    wid = sid * NCORES + lax.axis_index("c")
    row0 = wid * ROWS_PER_W
    bufs = (buf0, buf1)
    gsem = (gs0, gs1)
    wsem = (ws0, ws1)

    @pl.when(sid == 0)
    def _():
        pltpu.sync_copy(table_hbm, table_sp)

    iota16 = jnp.arange(16, dtype=jnp.int32)

    def stage_eighth(e, sync=False):
        r0 = wid * BATCH_PER_W + e * BH
        if sync:
            pltpu.sync_copy(a_hbm.at[pl.ds(r0, BH)], sa)
            pltpu.sync_copy(d_hbm.at[pl.ds(r0, BH)], sd)
            pltpu.sync_copy(s_hbm.at[pl.ds(r0, BH)], ss)
        else:
            pltpu.async_copy(a_hbm.at[pl.ds(r0, BH)], sa, ssem)
            pltpu.async_copy(d_hbm.at[pl.ds(r0, BH)], sd, ssem)
            pltpu.async_copy(s_hbm.at[pl.ds(r0, BH)], ss, ssem)

    def drain_stage():
        for buf in (sa, sd, ss):
            pltpu.make_async_copy(a_hbm.at[pl.ds(0, BH)], buf, ssem).wait()

    _COLS = tuple(range(0, L - 16, 16)) + (L - 16,)

    def compute_eighth(e):
        base = ((e // 2) % 2) * SLOT + (e % 2) * EIGHTH

        def crow(r, carry):
            for c in _COLS:
                av = sa[r, pl.ds(c, 16)]
                dv = sd[r, pl.ds(c, 16)]
                sv = ss[r, pl.ds(c, 16)]
                cc = av * (ND * NSTEP) + dv * NSTEP + sv
                cidx[pl.ds(base + r * L + c, 16)] = cc
            return carry

        lax.fori_loop(0, BH, crow, 0)

    def step_eighth(e):
        drain_stage()
        compute_eighth(e)
        if e + 1 < NE:
            stage_eighth(e + 1)

    def start_g(g, b):
        off = lax.rem(g * CH, 2 * SLOT)
        pltpu.async_copy(table_sp.at[cidx.at[pl.ds(off, CH)]],
                         bufs[b], gsem[b])

    def drain_g(b):
        pltpu.make_async_copy(out_hbm.at[pl.ds(0, CH)], bufs[b], gsem[b]).wait()

    def start_w(g, b):
        pltpu.async_copy(bufs[b], out_hbm.at[pl.ds(row0 + g * CH, CH)], wsem[b])

    def drain_w(b):
        pltpu.make_async_copy(bufs[b], out_hbm.at[pl.ds(0, CH)], wsem[b]).wait()

    stage_eighth(0, sync=True)
    compute_eighth(0)
    stage_eighth(1)
    plsc.subcore_barrier()

    start_g(0, 0)
    start_g(1, 1)
    step_eighth(1)
    drain_g(0)
    start_w(0, 0)

    def body(i, carry):
        drain_w(0)
        start_g(2 * i, 0)
        drain_g(1)
        start_w(2 * i - 1, 1)
        drain_w(1)
        start_g(2 * i + 1, 1)
        drain_g(0)
        start_w(2 * i, 0)
        return carry

    bounds = (1, 5, 9, 13, 19, 25, 31)
    for e in range(2, NE):
        lax.fori_loop(bounds[e - 2], bounds[e - 1], body, 0)
        step_eighth(e)
    lax.fori_loop(bounds[-1], PAIRS, body, 0)
    drain_g(1)
    start_w(2 * PAIRS - 1, 1)
    drain_w(0)
    drain_w(1)


def kernel(act_seq, act_dist, act_step, action_table, distance_table,
           step_table, gamma, beta):
    table = _build_table(action_table, distance_table, step_table, gamma, beta)
    out = _make_gather()(table, act_seq, act_dist, act_step)
    return out.reshape(B, L, H)
